# baseline (speedup 1.0000x reference)
import functools

import jax
import jax.numpy as jnp
from jax import lax
from jax.experimental import pallas as pl
from jax.experimental.pallas import tpu as pltpu
from jax.experimental.pallas import tpu_sc as plsc

N_NODES = 10000
LANES = 16
NC = 2
NS = 16
BN = 1000


def _mm_scale_body(dp_ref, x_ref, w_ref, g_ref, dv_ref):
    deg = jnp.sum(dp_ref[...], axis=0) + 1.0
    dinv = lax.rsqrt(deg)[:, None]
    h = jnp.dot(x_ref[...], w_ref[...], preferred_element_type=jnp.float32)
    g_ref[...] = h * dinv
    dv_ref[...] = dinv


def _tc_mm_scale(degp, x, w):
    nw = degp.shape[0]
    n, ci = x.shape
    co = w.shape[1]
    return pl.pallas_call(
        _mm_scale_body,
        grid=(1,),
        in_specs=[pl.BlockSpec((nw, n), lambda i: (0, 0)),
                  pl.BlockSpec((n, ci), lambda i: (0, 0)),
                  pl.BlockSpec((ci, co), lambda i: (0, 0))],
        out_specs=[pl.BlockSpec((n, co), lambda i: (0, 0)),
                   pl.BlockSpec((n, 1), lambda i: (0, 0))],
        out_shape=[jax.ShapeDtypeStruct((n, co), jnp.float32),
                   jax.ShapeDtypeStruct((n, 1), jnp.float32)],
    )(degp, x, w)


def _mid_body(s_ref, dv_ref, b_ref, w_ref, o_ref):
    s = s_ref[0] + s_ref[1]
    dv = dv_ref[...]
    z = jnp.maximum(s * dv + b_ref[...], 0.0)
    h2 = jnp.dot(z, w_ref[...], preferred_element_type=jnp.float32)
    o_ref[...] = h2 * dv


def _tc_mid(s1, dinv, b1, w2):
    _, n, c = s1.shape
    co = w2.shape[1]
    return pl.pallas_call(
        _mid_body,
        grid=(n // BN,),
        in_specs=[pl.BlockSpec((2, BN, c), lambda i: (0, i, 0)),
                  pl.BlockSpec((BN, 1), lambda i: (i, 0)),
                  pl.BlockSpec((1, c), lambda i: (0, 0)),
                  pl.BlockSpec((c, co), lambda i: (0, 0))],
        out_specs=pl.BlockSpec((BN, co), lambda i: (i, 0)),
        out_shape=jax.ShapeDtypeStruct((n, co), jnp.float32),
    )(s1, dinv, b1, w2)


def _fin_body(s_ref, dv_ref, b_ref, o_ref):
    o_ref[...] = (s_ref[0] + s_ref[1]) * dv_ref[...] + b_ref[...]


def _tc_final(s2, dinv, b2):
    _, n, c = s2.shape
    return pl.pallas_call(
        _fin_body,
        grid=(n // BN,),
        in_specs=[pl.BlockSpec((2, BN, c), lambda i: (0, i, 0)),
                  pl.BlockSpec((BN, 1), lambda i: (i, 0)),
                  pl.BlockSpec((1, c), lambda i: (0, 0))],
        out_specs=pl.BlockSpec((BN, c), lambda i: (i, 0)),
        out_shape=jax.ShapeDtypeStruct((n, c), jnp.float32),
    )(s2, dinv, b2)


def _sc_deg(dst):
    e = dst.shape[0]
    nw = NC * NS
    epw = e // nw
    n = N_NODES
    mesh = plsc.VectorSubcoreMesh(core_axis_name="c", subcore_axis_name="s")

    @functools.partial(
        pl.kernel,
        out_type=jax.ShapeDtypeStruct((nw, n), jnp.float32),
        mesh=mesh,
        scratch_types=[pltpu.VMEM((n,), jnp.float32),
                       pltpu.VMEM((epw,), jnp.int32)],
        compiler_params=pltpu.CompilerParams(needs_layout_passes=False),
    )
    def k(dst_hbm, out_hbm, deg_v, idx_v):
        cid = lax.axis_index("c")
        sid = lax.axis_index("s")
        wid = sid * NC + cid

        def zero(i, carry):
            deg_v[pl.ds(i * LANES, LANES)] = jnp.zeros((LANES,), jnp.float32)
            return carry
        lax.fori_loop(0, n // LANES, zero, 0)

        pltpu.sync_copy(dst_hbm.at[pl.ds(wid * epw, epw)], idx_v)
        ones = jnp.full((LANES,), 1.0, jnp.float32)

        def body(j, carry):
            idx = idx_v[pl.ds(j * LANES, LANES)]
            plsc.addupdate_scatter(deg_v, [idx], ones)
            return carry
        lax.fori_loop(0, epw // LANES, body, 0)

        pltpu.sync_copy(deg_v, out_hbm.at[wid])

    return k(dst)


def _sc_edge(g, src2, dst2, zslice):
    n, c = g.shape
    nrows = src2.shape[0]
    bch = src2.shape[1]
    nw = NC * NS
    nrt = nrows // nw
    nlft = nrows - nrt * nw
    rpt = n // NS
    half = (nrt + 1) // 2
    seg0 = half + (half & 1)
    seg1 = nrt - seg0
    mesh = plsc.VectorSubcoreMesh(core_axis_name="c", subcore_axis_name="s")

    @functools.partial(
        pl.kernel,
        out_type=jax.ShapeDtypeStruct((NC, n, c), jnp.float32),
        mesh=mesh,
        scratch_types=[
            pltpu.VMEM_SHARED((n, c), jnp.float32),
            pltpu.VMEM((seg0, bch), jnp.int32),
            pltpu.VMEM((seg0, bch), jnp.int32),
            pltpu.VMEM((2, bch, c), jnp.float32),
            pltpu.VMEM((1, bch), jnp.int32),
            pltpu.VMEM((1, bch), jnp.int32),
            pltpu.SemaphoreType.DMA,
            pltpu.SemaphoreType.DMA,
            pltpu.SemaphoreType.DMA,
            pltpu.SemaphoreType.DMA,
        ],
        compiler_params=pltpu.CompilerParams(use_tc_tiling_on_sc=False),
    )
    def k(g_hbm, src_hbm, dst_hbm, z_hbm, out_hbm,
          acc, sidx, didx, rows, lsidx, ldidx, sem0, sem1, ssem0, ssem1):
        cid = lax.axis_index("c")
        sid = lax.axis_index("s")
        wid = cid * NS + sid
        r0 = sid * rpt

        @pl.when(cid == 0)
        def _():
            pltpu.sync_copy(g_hbm.at[pl.ds(r0, rpt)], acc.at[pl.ds(r0, rpt)])

        @pl.when(cid != 0)
        def _():
            pltpu.sync_copy(z_hbm, acc.at[pl.ds(r0, rpt)])

        plsc.subcore_barrier()

        i0 = wid * nrt

        def run_segment(row0, m):
            pltpu.sync_copy(src_hbm.at[pl.ds(row0, m)], sidx.at[pl.ds(0, m)])
            pltpu.sync_copy(dst_hbm.at[pl.ds(row0, m)], didx.at[pl.ds(0, m)])
            pltpu.async_copy(g_hbm.at[sidx.at[0]], rows.at[0], sem0)
            pltpu.async_copy(g_hbm.at[sidx.at[1]], rows.at[1], sem1)

            def step(t, carry):
                a = 2 * t
                b = a + 1
                pltpu.make_async_copy(g_hbm.at[sidx.at[a]],
                                      rows.at[0], sem0).wait()
                pltpu.async_copy(rows.at[0], acc.at[didx.at[a]], ssem0,
                                 add=True)
                pltpu.make_async_copy(g_hbm.at[sidx.at[b]],
                                      rows.at[1], sem1).wait()
                pltpu.async_copy(rows.at[1], acc.at[didx.at[b]], ssem1,
                                 add=True)

                @pl.when(a + 2 < m)
                def _():
                    pltpu.make_async_copy(rows.at[0], acc.at[didx.at[a]],
                                          ssem0).wait()
                    pltpu.async_copy(g_hbm.at[sidx.at[a + 2]],
                                     rows.at[0], sem0)
                    pltpu.make_async_copy(rows.at[1], acc.at[didx.at[b]],
                                          ssem1).wait()
                    pltpu.async_copy(g_hbm.at[sidx.at[b + 2]],
                                     rows.at[1], sem1)
                return carry
            lax.fori_loop(0, m // 2, step, 0)

            pltpu.make_async_copy(rows.at[0], acc.at[didx.at[m - 2]],
                                  ssem0).wait()
            pltpu.make_async_copy(rows.at[1], acc.at[didx.at[m - 1]],
                                  ssem1).wait()

        run_segment(i0, seg0)
        if seg1 >= 2:
            run_segment(i0 + seg0, seg1 - (seg1 & 1))
        if seg1 & 1:
            a = nrt - 1
            pltpu.sync_copy(src_hbm.at[pl.ds(i0 + a, 1)], lsidx)
            pltpu.sync_copy(dst_hbm.at[pl.ds(i0 + a, 1)], ldidx)
            pltpu.async_copy(g_hbm.at[lsidx.at[0]], rows.at[0], sem0).wait()
            pltpu.sync_copy(rows.at[0], acc.at[ldidx.at[0]], add=True)

        if nlft:
            @pl.when(wid < nlft)
            def _():
                j0 = nw * nrt + wid
                pltpu.sync_copy(src_hbm.at[pl.ds(j0, 1)], lsidx)
                pltpu.sync_copy(dst_hbm.at[pl.ds(j0, 1)], ldidx)
                pltpu.async_copy(g_hbm.at[lsidx.at[0]],
                                 rows.at[0], sem0).wait()
                pltpu.sync_copy(rows.at[0], acc.at[ldidx.at[0]], add=True)

        plsc.subcore_barrier()
        pltpu.sync_copy(acc.at[pl.ds(r0, rpt)],
                        out_hbm.at[cid, pl.ds(r0, rpt)])

    return k(g, src2, dst2, zslice)


def kernel(x, edge_index, W1, b1, W2, b2):
    ei = edge_index.astype(jnp.int32)
    src = ei[0]
    dst = ei[1]
    e = src.shape[0]
    src2 = src.reshape(e // 128, 128)
    dst2 = dst.reshape(e // 128, 128)

    degp = _sc_deg(dst)
    g1, dinv = _tc_mm_scale(degp, x, W1)

    z1 = jnp.zeros((N_NODES // NS, g1.shape[1]), jnp.float32)
    s1 = _sc_edge(g1, src2, dst2, z1)
    g2 = _tc_mid(s1, dinv, b1.reshape(1, -1), W2)

    z2 = jnp.zeros((N_NODES // NS, g2.shape[1]), jnp.float32)
    s2 = _sc_edge(g2, src2, dst2, z2)
    return _tc_final(s2, dinv, b2.reshape(1, -1))


# speedup vs baseline: 1.1889x; 1.1889x over previous
import functools

import jax
import jax.numpy as jnp
from jax import lax
from jax.experimental import pallas as pl
from jax.experimental.pallas import tpu as pltpu
from jax.experimental.pallas import tpu_sc as plsc

N_NODES = 10000
LANES = 16
NC = 2
NS = 16
BN = 1000


def _mm_scale_body(dp_ref, x_ref, w_ref, g_ref, dv_ref):
    deg = jnp.sum(dp_ref[...], axis=0) + 1.0
    dinv = lax.rsqrt(deg)[:, None]
    h = jnp.dot(x_ref[...], w_ref[...], preferred_element_type=jnp.float32)
    g_ref[...] = h * dinv
    dv_ref[...] = dinv


def _tc_mm_scale(degp, x, w):
    nw = degp.shape[0]
    n, ci = x.shape
    co = w.shape[1]
    return pl.pallas_call(
        _mm_scale_body,
        grid=(1,),
        in_specs=[pl.BlockSpec((nw, n), lambda i: (0, 0)),
                  pl.BlockSpec((n, ci), lambda i: (0, 0)),
                  pl.BlockSpec((ci, co), lambda i: (0, 0))],
        out_specs=[pl.BlockSpec((n, co), lambda i: (0, 0)),
                   pl.BlockSpec((n, 1), lambda i: (0, 0))],
        out_shape=[jax.ShapeDtypeStruct((n, co), jnp.float32),
                   jax.ShapeDtypeStruct((n, 1), jnp.float32)],
    )(degp, x, w)


def _mid_body(s_ref, dv_ref, b_ref, w_ref, o_ref):
    s = s_ref[0] + s_ref[1]
    dv = dv_ref[...]
    z = jnp.maximum(s * dv + b_ref[...], 0.0)
    h2 = jnp.dot(z, w_ref[...], preferred_element_type=jnp.float32)
    o_ref[...] = h2 * dv


def _tc_mid(s1, dinv, b1, w2):
    _, n, c = s1.shape
    co = w2.shape[1]
    return pl.pallas_call(
        _mid_body,
        grid=(n // BN,),
        in_specs=[pl.BlockSpec((2, BN, c), lambda i: (0, i, 0)),
                  pl.BlockSpec((BN, 1), lambda i: (i, 0)),
                  pl.BlockSpec((1, c), lambda i: (0, 0)),
                  pl.BlockSpec((c, co), lambda i: (0, 0))],
        out_specs=pl.BlockSpec((BN, co), lambda i: (i, 0)),
        out_shape=jax.ShapeDtypeStruct((n, co), jnp.float32),
    )(s1, dinv, b1, w2)


def _fin_body(s_ref, dv_ref, b_ref, o_ref):
    o_ref[...] = (s_ref[0] + s_ref[1]) * dv_ref[...] + b_ref[...]


def _tc_final(s2, dinv, b2):
    _, n, c = s2.shape
    return pl.pallas_call(
        _fin_body,
        grid=(n // BN,),
        in_specs=[pl.BlockSpec((2, BN, c), lambda i: (0, i, 0)),
                  pl.BlockSpec((BN, 1), lambda i: (i, 0)),
                  pl.BlockSpec((1, c), lambda i: (0, 0))],
        out_specs=pl.BlockSpec((BN, c), lambda i: (i, 0)),
        out_shape=jax.ShapeDtypeStruct((n, c), jnp.float32),
    )(s2, dinv, b2)


def _sc_deg(dst):
    e = dst.shape[0]
    nw = NC * NS
    epw = e // nw
    n = N_NODES
    mesh = plsc.VectorSubcoreMesh(core_axis_name="c", subcore_axis_name="s")

    @functools.partial(
        pl.kernel,
        out_type=jax.ShapeDtypeStruct((nw, n), jnp.float32),
        mesh=mesh,
        scratch_types=[pltpu.VMEM((n,), jnp.float32),
                       pltpu.VMEM((epw,), jnp.int32)],
        compiler_params=pltpu.CompilerParams(needs_layout_passes=False),
    )
    def k(dst_hbm, out_hbm, deg_v, idx_v):
        cid = lax.axis_index("c")
        sid = lax.axis_index("s")
        wid = sid * NC + cid

        def zero(i, carry):
            deg_v[pl.ds(i * LANES, LANES)] = jnp.zeros((LANES,), jnp.float32)
            return carry
        lax.fori_loop(0, n // LANES, zero, 0)

        pltpu.sync_copy(dst_hbm.at[pl.ds(wid * epw, epw)], idx_v)
        ones = jnp.full((LANES,), 1.0, jnp.float32)

        def body(j, carry):
            idx = idx_v[pl.ds(j * LANES, LANES)]
            plsc.addupdate_scatter(deg_v, [idx], ones)
            return carry
        lax.fori_loop(0, epw // LANES, body, 0)

        pltpu.sync_copy(deg_v, out_hbm.at[wid])

    return k(dst)


def _sc_edge(g, src2, dst2, zslice):
    n, c = g.shape
    nrows = src2.shape[0]
    bch = src2.shape[1]
    nw = NC * NS
    nrt = nrows // nw
    nlft = nrows - nrt * nw
    rpt = n // NS
    half = (nrt + 1) // 2
    seg0 = half + (half & 1)
    seg1 = nrt - seg0
    mesh = plsc.VectorSubcoreMesh(core_axis_name="c", subcore_axis_name="s")

    @functools.partial(
        pl.kernel,
        out_type=jax.ShapeDtypeStruct((NC, n, c), jnp.float32),
        mesh=mesh,
        scratch_types=[
            pltpu.VMEM_SHARED((n, c), jnp.float32),
            pltpu.VMEM((seg0, bch), jnp.int32),
            pltpu.VMEM((seg0, bch), jnp.int32),
            pltpu.VMEM((2, bch, c), jnp.float32),
            pltpu.VMEM((1, bch), jnp.int32),
            pltpu.VMEM((1, bch), jnp.int32),
            pltpu.SemaphoreType.DMA,
            pltpu.SemaphoreType.DMA,
        ],
        compiler_params=pltpu.CompilerParams(use_tc_tiling_on_sc=False),
    )
    def k(g_hbm, src_hbm, dst_hbm, z_hbm, out_hbm,
          acc, sidx, didx, rows, lsidx, ldidx, sem0, sem1):
        cid = lax.axis_index("c")
        sid = lax.axis_index("s")
        wid = cid * NS + sid
        r0 = sid * rpt

        @pl.when(cid == 0)
        def _():
            pltpu.sync_copy(g_hbm.at[pl.ds(r0, rpt)], acc.at[pl.ds(r0, rpt)])

        @pl.when(cid != 0)
        def _():
            pltpu.sync_copy(z_hbm, acc.at[pl.ds(r0, rpt)])

        plsc.subcore_barrier()

        i0 = wid * nrt

        gbytes = bch * c * 4

        def run_segment(row0, m):
            pltpu.sync_copy(src_hbm.at[pl.ds(row0, m)], sidx.at[pl.ds(0, m)])
            pltpu.sync_copy(dst_hbm.at[pl.ds(row0, m)], didx.at[pl.ds(0, m)])
            pltpu.async_copy(g_hbm.at[sidx.at[0]], rows.at[0], sem0)
            pltpu.async_copy(g_hbm.at[sidx.at[1]], rows.at[1], sem1)

            def step(t, carry):
                a = 2 * t
                b = a + 1
                pltpu.make_async_copy(g_hbm.at[pl.ds(0, bch)],
                                      rows.at[0], sem0).wait()
                pltpu.sync_copy(rows.at[0], acc.at[didx.at[a]], add=True)

                @pl.when(a + 2 < m)
                def _():
                    pltpu.async_copy(g_hbm.at[sidx.at[a + 2]],
                                     rows.at[0], sem0)

                pltpu.make_async_copy(g_hbm.at[pl.ds(0, bch)],
                                      rows.at[1], sem1).wait()
                pltpu.sync_copy(rows.at[1], acc.at[didx.at[b]], add=True)

                @pl.when(b + 2 < m)
                def _():
                    pltpu.async_copy(g_hbm.at[sidx.at[b + 2]],
                                     rows.at[1], sem1)
                return carry
            lax.fori_loop(0, m // 2, step, 0)

        run_segment(i0, seg0)
        if seg1 >= 2:
            run_segment(i0 + seg0, seg1 - (seg1 & 1))
        if seg1 & 1:
            a = nrt - 1
            pltpu.sync_copy(src_hbm.at[pl.ds(i0 + a, 1)], lsidx)
            pltpu.sync_copy(dst_hbm.at[pl.ds(i0 + a, 1)], ldidx)
            pltpu.async_copy(g_hbm.at[lsidx.at[0]], rows.at[0], sem0)
            pltpu.make_async_copy(g_hbm.at[pl.ds(0, bch)],
                                  rows.at[0], sem0).wait()
            pltpu.sync_copy(rows.at[0], acc.at[ldidx.at[0]], add=True)

        if nlft:
            @pl.when(wid < nlft)
            def _():
                j0 = nw * nrt + wid
                pltpu.sync_copy(src_hbm.at[pl.ds(j0, 1)], lsidx)
                pltpu.sync_copy(dst_hbm.at[pl.ds(j0, 1)], ldidx)
                pltpu.async_copy(g_hbm.at[lsidx.at[0]], rows.at[0], sem0)
                pltpu.make_async_copy(g_hbm.at[pl.ds(0, bch)],
                                      rows.at[0], sem0).wait()
                pltpu.sync_copy(rows.at[0], acc.at[ldidx.at[0]], add=True)

        plsc.subcore_barrier()
        pltpu.sync_copy(acc.at[pl.ds(r0, rpt)],
                        out_hbm.at[cid, pl.ds(r0, rpt)])

    return k(g, src2, dst2, zslice)


def kernel(x, edge_index, W1, b1, W2, b2):
    ei = edge_index.astype(jnp.int32)
    src = ei[0]
    dst = ei[1]
    e = src.shape[0]
    src2 = src.reshape(e // 128, 128)
    dst2 = dst.reshape(e // 128, 128)

    degp = _sc_deg(dst)
    g1, dinv = _tc_mm_scale(degp, x, W1)

    z1 = jnp.zeros((N_NODES // NS, g1.shape[1]), jnp.float32)
    s1 = _sc_edge(g1, src2, dst2, z1)
    g2 = _tc_mid(s1, dinv, b1.reshape(1, -1), W2)

    z2 = jnp.zeros((N_NODES // NS, g2.shape[1]), jnp.float32)
    s2 = _sc_edge(g2, src2, dst2, z2)
    return _tc_final(s2, dinv, b2.reshape(1, -1))


# speedup vs baseline: 1.3335x; 1.1216x over previous
import functools

import jax
import jax.numpy as jnp
from jax import lax
from jax.experimental import pallas as pl
from jax.experimental.pallas import tpu as pltpu
from jax.experimental.pallas import tpu_sc as plsc

N_NODES = 10000
LANES = 16
NC = 2
NS = 16
BN = 1000


def _mm_scale_body(dp_ref, x_ref, w_ref, g_ref, dv_ref):
    deg = jnp.sum(dp_ref[...], axis=0) + 1.0
    dinv = lax.rsqrt(deg)[:, None]
    h = jnp.dot(x_ref[...], w_ref[...], preferred_element_type=jnp.float32)
    g_ref[...] = (h * dinv).astype(jnp.bfloat16)
    dv_ref[...] = dinv


def _tc_mm_scale(degp, x, w):
    nw = degp.shape[0]
    n, ci = x.shape
    co = w.shape[1]
    return pl.pallas_call(
        _mm_scale_body,
        grid=(1,),
        in_specs=[pl.BlockSpec((nw, n), lambda i: (0, 0)),
                  pl.BlockSpec((n, ci), lambda i: (0, 0)),
                  pl.BlockSpec((ci, co), lambda i: (0, 0))],
        out_specs=[pl.BlockSpec((n, co), lambda i: (0, 0)),
                   pl.BlockSpec((n, 1), lambda i: (0, 0))],
        out_shape=[jax.ShapeDtypeStruct((n, co), jnp.bfloat16),
                   jax.ShapeDtypeStruct((n, 1), jnp.float32)],
    )(degp, x, w)


def _mid_body(s_ref, dv_ref, b_ref, w_ref, o_ref):
    s = s_ref[0].astype(jnp.float32) + s_ref[1].astype(jnp.float32)
    dv = dv_ref[...]
    z = jnp.maximum(s * dv + b_ref[...], 0.0)
    h2 = jnp.dot(z, w_ref[...], preferred_element_type=jnp.float32)
    o_ref[...] = (h2 * dv).astype(jnp.bfloat16)


def _tc_mid(s1, dinv, b1, w2):
    _, n, c = s1.shape
    co = w2.shape[1]
    return pl.pallas_call(
        _mid_body,
        grid=(n // BN,),
        in_specs=[pl.BlockSpec((2, BN, c), lambda i: (0, i, 0)),
                  pl.BlockSpec((BN, 1), lambda i: (i, 0)),
                  pl.BlockSpec((1, c), lambda i: (0, 0)),
                  pl.BlockSpec((c, co), lambda i: (0, 0))],
        out_specs=pl.BlockSpec((BN, co), lambda i: (i, 0)),
        out_shape=jax.ShapeDtypeStruct((n, co), jnp.bfloat16),
    )(s1, dinv, b1, w2)


def _fin_body(s_ref, dv_ref, b_ref, o_ref):
    s = s_ref[0].astype(jnp.float32) + s_ref[1].astype(jnp.float32)
    o_ref[...] = s * dv_ref[...] + b_ref[...]


def _tc_final(s2, dinv, b2):
    _, n, c = s2.shape
    return pl.pallas_call(
        _fin_body,
        grid=(n // BN,),
        in_specs=[pl.BlockSpec((2, BN, c), lambda i: (0, i, 0)),
                  pl.BlockSpec((BN, 1), lambda i: (i, 0)),
                  pl.BlockSpec((1, c), lambda i: (0, 0))],
        out_specs=pl.BlockSpec((BN, c), lambda i: (i, 0)),
        out_shape=jax.ShapeDtypeStruct((n, c), jnp.float32),
    )(s2, dinv, b2)


def _sc_deg(dst):
    e = dst.shape[0]
    nw = NC * NS
    epw = e // nw
    n = N_NODES
    mesh = plsc.VectorSubcoreMesh(core_axis_name="c", subcore_axis_name="s")

    @functools.partial(
        pl.kernel,
        out_type=jax.ShapeDtypeStruct((nw, n), jnp.float32),
        mesh=mesh,
        scratch_types=[pltpu.VMEM((n,), jnp.float32),
                       pltpu.VMEM((epw,), jnp.int32)],
        compiler_params=pltpu.CompilerParams(needs_layout_passes=False),
    )
    def k(dst_hbm, out_hbm, deg_v, idx_v):
        cid = lax.axis_index("c")
        sid = lax.axis_index("s")
        wid = sid * NC + cid

        def zero(i, carry):
            deg_v[pl.ds(i * LANES, LANES)] = jnp.zeros((LANES,), jnp.float32)
            return carry
        lax.fori_loop(0, n // LANES, zero, 0)

        pltpu.sync_copy(dst_hbm.at[pl.ds(wid * epw, epw)], idx_v)
        ones = jnp.full((LANES,), 1.0, jnp.float32)

        def body(j, carry):
            idx = idx_v[pl.ds(j * LANES, LANES)]
            plsc.addupdate_scatter(deg_v, [idx], ones)
            return carry
        lax.fori_loop(0, epw // LANES, body, 0)

        pltpu.sync_copy(deg_v, out_hbm.at[wid])

    return k(dst)


def _sc_edge(g, src2, dst2, zslice):
    n, c = g.shape
    nrows = src2.shape[0]
    bch = src2.shape[1]
    nw = NC * NS
    nrt = nrows // nw
    nlft = nrows - nrt * nw
    rpt = n // NS
    esz = jnp.dtype(g.dtype).itemsize
    acc_words = n * c * esz // 4
    def _words(seg):
        return acc_words + NS * (2 * seg * bch + bch * c * esz // 2 + 128)
    if _words(nrt) <= 2_000_000:
        seg0 = nrt - (nrt & 1)
    else:
        half = (nrt + 1) // 2
        seg0 = half + (half & 1)
    seg1 = nrt - seg0
    mesh = plsc.VectorSubcoreMesh(core_axis_name="c", subcore_axis_name="s")

    @functools.partial(
        pl.kernel,
        out_type=jax.ShapeDtypeStruct((NC, n, c), g.dtype),
        mesh=mesh,
        scratch_types=[
            pltpu.VMEM_SHARED((n, c), g.dtype),
            pltpu.VMEM((seg0, bch), jnp.int32),
            pltpu.VMEM((seg0, bch), jnp.int32),
            pltpu.VMEM((2, bch, c), g.dtype),
            pltpu.VMEM((1, bch), jnp.int32),
            pltpu.VMEM((1, bch), jnp.int32),
            pltpu.SemaphoreType.DMA,
            pltpu.SemaphoreType.DMA,
        ],
        compiler_params=pltpu.CompilerParams(use_tc_tiling_on_sc=False),
    )
    def k(g_hbm, src_hbm, dst_hbm, z_hbm, out_hbm,
          acc, sidx, didx, rows, lsidx, ldidx, sem0, sem1):
        cid = lax.axis_index("c")
        sid = lax.axis_index("s")
        wid = cid * NS + sid
        r0 = sid * rpt

        @pl.when(cid == 0)
        def _():
            pltpu.sync_copy(g_hbm.at[pl.ds(r0, rpt)], acc.at[pl.ds(r0, rpt)])

        @pl.when(cid != 0)
        def _():
            pltpu.sync_copy(z_hbm, acc.at[pl.ds(r0, rpt)])

        plsc.subcore_barrier()

        i0 = wid * nrt

        gbytes = bch * c * esz

        def run_segment(row0, m):
            pltpu.sync_copy(src_hbm.at[pl.ds(row0, m)], sidx.at[pl.ds(0, m)])
            pltpu.sync_copy(dst_hbm.at[pl.ds(row0, m)], didx.at[pl.ds(0, m)])
            pltpu.async_copy(g_hbm.at[sidx.at[0]], rows.at[0], sem0)
            pltpu.async_copy(g_hbm.at[sidx.at[1]], rows.at[1], sem1)

            def step(t, carry):
                a = 2 * t
                b = a + 1
                pltpu.make_async_copy(g_hbm.at[pl.ds(0, bch)],
                                      rows.at[0], sem0).wait()
                pltpu.sync_copy(rows.at[0], acc.at[didx.at[a]], add=True)

                @pl.when(a + 2 < m)
                def _():
                    pltpu.async_copy(g_hbm.at[sidx.at[a + 2]],
                                     rows.at[0], sem0)

                pltpu.make_async_copy(g_hbm.at[pl.ds(0, bch)],
                                      rows.at[1], sem1).wait()
                pltpu.sync_copy(rows.at[1], acc.at[didx.at[b]], add=True)

                @pl.when(b + 2 < m)
                def _():
                    pltpu.async_copy(g_hbm.at[sidx.at[b + 2]],
                                     rows.at[1], sem1)
                return carry
            lax.fori_loop(0, m // 2, step, 0)

        run_segment(i0, seg0)
        if seg1 >= 2:
            run_segment(i0 + seg0, seg1 - (seg1 & 1))
        if seg1 & 1:
            a = nrt - 1
            pltpu.sync_copy(src_hbm.at[pl.ds(i0 + a, 1)], lsidx)
            pltpu.sync_copy(dst_hbm.at[pl.ds(i0 + a, 1)], ldidx)
            pltpu.async_copy(g_hbm.at[lsidx.at[0]], rows.at[0], sem0)
            pltpu.make_async_copy(g_hbm.at[pl.ds(0, bch)],
                                  rows.at[0], sem0).wait()
            pltpu.sync_copy(rows.at[0], acc.at[ldidx.at[0]], add=True)

        if nlft:
            @pl.when(wid < nlft)
            def _():
                j0 = nw * nrt + wid
                pltpu.sync_copy(src_hbm.at[pl.ds(j0, 1)], lsidx)
                pltpu.sync_copy(dst_hbm.at[pl.ds(j0, 1)], ldidx)
                pltpu.async_copy(g_hbm.at[lsidx.at[0]], rows.at[0], sem0)
                pltpu.make_async_copy(g_hbm.at[pl.ds(0, bch)],
                                      rows.at[0], sem0).wait()
                pltpu.sync_copy(rows.at[0], acc.at[ldidx.at[0]], add=True)

        plsc.subcore_barrier()
        pltpu.sync_copy(acc.at[pl.ds(r0, rpt)],
                        out_hbm.at[cid, pl.ds(r0, rpt)])

    return k(g, src2, dst2, zslice)


def kernel(x, edge_index, W1, b1, W2, b2):
    ei = edge_index.astype(jnp.int32)
    src = ei[0]
    dst = ei[1]
    e = src.shape[0]
    src2 = src.reshape(e // 128, 128)
    dst2 = dst.reshape(e // 128, 128)

    degp = _sc_deg(dst)
    g1, dinv = _tc_mm_scale(degp, x, W1)

    z1 = jnp.zeros((N_NODES // NS, g1.shape[1]), g1.dtype)
    s1 = _sc_edge(g1, src2, dst2, z1)
    g2 = _tc_mid(s1, dinv, b1.reshape(1, -1), W2)

    z2 = jnp.zeros((N_NODES // NS, g2.shape[1]), g2.dtype)
    s2 = _sc_edge(g2, src2, dst2, z2)
    return _tc_final(s2, dinv, b2.reshape(1, -1))


# speedup vs baseline: 1.3694x; 1.0270x over previous
import functools

import jax
import jax.numpy as jnp
from jax import lax
from jax.experimental import pallas as pl
from jax.experimental.pallas import tpu as pltpu
from jax.experimental.pallas import tpu_sc as plsc

N_NODES = 10000
LANES = 16
NC = 2
NS = 16
BN = 1000


def _mm_scale_body(dp_ref, x_ref, w_ref, g_ref, dv_ref):
    deg = jnp.sum(dp_ref[...], axis=0) + 1.0
    dinv = lax.rsqrt(deg)[:, None]
    h = jnp.dot(x_ref[...], w_ref[...], preferred_element_type=jnp.float32)
    g_ref[...] = (h * dinv).astype(jnp.bfloat16)
    dv_ref[...] = dinv


def _tc_mm_scale(degp, x, w):
    nw = degp.shape[0]
    n, ci = x.shape
    co = w.shape[1]
    return pl.pallas_call(
        _mm_scale_body,
        grid=(1,),
        in_specs=[pl.BlockSpec((nw, n), lambda i: (0, 0)),
                  pl.BlockSpec((n, ci), lambda i: (0, 0)),
                  pl.BlockSpec((ci, co), lambda i: (0, 0))],
        out_specs=[pl.BlockSpec((n, co), lambda i: (0, 0)),
                   pl.BlockSpec((n, 1), lambda i: (0, 0))],
        out_shape=[jax.ShapeDtypeStruct((n, co), jnp.bfloat16),
                   jax.ShapeDtypeStruct((n, 1), jnp.float32)],
    )(degp, x, w)


def _mid_body(s_ref, dv_ref, b_ref, w_ref, o_ref):
    s = s_ref[0].astype(jnp.float32) + s_ref[1].astype(jnp.float32)
    dv = dv_ref[...]
    z = jnp.maximum(s * dv + b_ref[...], 0.0)
    h2 = jnp.dot(z, w_ref[...], preferred_element_type=jnp.float32)
    o_ref[...] = (h2 * dv).astype(jnp.bfloat16)


def _tc_mid(s1, dinv, b1, w2):
    _, n, c = s1.shape
    co = w2.shape[1]
    return pl.pallas_call(
        _mid_body,
        grid=(1,),
        in_specs=[pl.BlockSpec((2, n, c), lambda i: (0, 0, 0)),
                  pl.BlockSpec((n, 1), lambda i: (0, 0)),
                  pl.BlockSpec((1, c), lambda i: (0, 0)),
                  pl.BlockSpec((c, co), lambda i: (0, 0))],
        out_specs=pl.BlockSpec((n, co), lambda i: (0, 0)),
        out_shape=jax.ShapeDtypeStruct((n, co), jnp.bfloat16),
    )(s1, dinv, b1, w2)


def _fin_body(s_ref, dv_ref, b_ref, o_ref):
    s = s_ref[0].astype(jnp.float32) + s_ref[1].astype(jnp.float32)
    o_ref[...] = s * dv_ref[...] + b_ref[...]


def _tc_final(s2, dinv, b2):
    _, n, c = s2.shape
    return pl.pallas_call(
        _fin_body,
        grid=(1,),
        in_specs=[pl.BlockSpec((2, n, c), lambda i: (0, 0, 0)),
                  pl.BlockSpec((n, 1), lambda i: (0, 0)),
                  pl.BlockSpec((1, c), lambda i: (0, 0))],
        out_specs=pl.BlockSpec((n, c), lambda i: (0, 0)),
        out_shape=jax.ShapeDtypeStruct((n, c), jnp.float32),
    )(s2, dinv, b2)


def _sc_deg(dst):
    e = dst.shape[0]
    nw = NC * NS
    epw = e // nw
    n = N_NODES
    mesh = plsc.VectorSubcoreMesh(core_axis_name="c", subcore_axis_name="s")

    @functools.partial(
        pl.kernel,
        out_type=jax.ShapeDtypeStruct((nw, n), jnp.float32),
        mesh=mesh,
        scratch_types=[pltpu.VMEM((n,), jnp.float32),
                       pltpu.VMEM((epw,), jnp.int32)],
        compiler_params=pltpu.CompilerParams(needs_layout_passes=False),
    )
    def k(dst_hbm, out_hbm, deg_v, idx_v):
        cid = lax.axis_index("c")
        sid = lax.axis_index("s")
        wid = sid * NC + cid

        def zero(i, carry):
            deg_v[pl.ds(i * LANES, LANES)] = jnp.zeros((LANES,), jnp.float32)
            return carry
        lax.fori_loop(0, n // LANES, zero, 0)

        pltpu.sync_copy(dst_hbm.at[pl.ds(wid * epw, epw)], idx_v)
        ones = jnp.full((LANES,), 1.0, jnp.float32)

        def body(j, carry):
            idx = idx_v[pl.ds(j * LANES, LANES)]
            plsc.addupdate_scatter(deg_v, [idx], ones)
            return carry
        lax.fori_loop(0, epw // LANES, body, 0)

        pltpu.sync_copy(deg_v, out_hbm.at[wid])

    return k(dst)


def _sc_edge(g, src2, dst2, zslice):
    n, c = g.shape
    nrows = src2.shape[0]
    bch = src2.shape[1]
    nw = NC * NS
    nrt = nrows // nw
    nlft = nrows - nrt * nw
    rpt = n // NS
    esz = jnp.dtype(g.dtype).itemsize
    acc_words = n * c * esz // 4
    def _words(seg):
        return acc_words + NS * (2 * seg * bch + bch * c * esz // 2 + 128)
    if _words(nrt) <= 2_000_000:
        seg0 = nrt - (nrt & 1)
    else:
        half = (nrt + 1) // 2
        seg0 = half + (half & 1)
    seg1 = nrt - seg0
    mesh = plsc.VectorSubcoreMesh(core_axis_name="c", subcore_axis_name="s")

    @functools.partial(
        pl.kernel,
        out_type=jax.ShapeDtypeStruct((NC, n, c), g.dtype),
        mesh=mesh,
        scratch_types=[
            pltpu.VMEM_SHARED((n, c), g.dtype),
            pltpu.VMEM((seg0, bch), jnp.int32),
            pltpu.VMEM((seg0, bch), jnp.int32),
            pltpu.VMEM((2, bch, c), g.dtype),
            pltpu.VMEM((1, bch), jnp.int32),
            pltpu.VMEM((1, bch), jnp.int32),
            pltpu.SemaphoreType.DMA,
            pltpu.SemaphoreType.DMA,
        ],
        compiler_params=pltpu.CompilerParams(use_tc_tiling_on_sc=False),
    )
    def k(g_hbm, src_hbm, dst_hbm, z_hbm, out_hbm,
          acc, sidx, didx, rows, lsidx, ldidx, sem0, sem1):
        cid = lax.axis_index("c")
        sid = lax.axis_index("s")
        wid = cid * NS + sid
        r0 = sid * rpt

        @pl.when(cid == 0)
        def _():
            pltpu.sync_copy(g_hbm.at[pl.ds(r0, rpt)], acc.at[pl.ds(r0, rpt)])

        @pl.when(cid != 0)
        def _():
            pltpu.sync_copy(z_hbm, acc.at[pl.ds(r0, rpt)])

        plsc.subcore_barrier()

        i0 = wid * nrt

        gbytes = bch * c * esz

        def run_segment(row0, m):
            pltpu.sync_copy(src_hbm.at[pl.ds(row0, m)], sidx.at[pl.ds(0, m)])
            pltpu.sync_copy(dst_hbm.at[pl.ds(row0, m)], didx.at[pl.ds(0, m)])
            pltpu.async_copy(g_hbm.at[sidx.at[0]], rows.at[0], sem0)
            pltpu.async_copy(g_hbm.at[sidx.at[1]], rows.at[1], sem1)

            def step(t, carry):
                a = 2 * t
                b = a + 1
                pltpu.make_async_copy(g_hbm.at[pl.ds(0, bch)],
                                      rows.at[0], sem0).wait()
                pltpu.sync_copy(rows.at[0], acc.at[didx.at[a]], add=True)

                @pl.when(a + 2 < m)
                def _():
                    pltpu.async_copy(g_hbm.at[sidx.at[a + 2]],
                                     rows.at[0], sem0)

                pltpu.make_async_copy(g_hbm.at[pl.ds(0, bch)],
                                      rows.at[1], sem1).wait()
                pltpu.sync_copy(rows.at[1], acc.at[didx.at[b]], add=True)

                @pl.when(b + 2 < m)
                def _():
                    pltpu.async_copy(g_hbm.at[sidx.at[b + 2]],
                                     rows.at[1], sem1)
                return carry
            lax.fori_loop(0, m // 2, step, 0)

        run_segment(i0, seg0)
        if seg1 >= 2:
            run_segment(i0 + seg0, seg1 - (seg1 & 1))
        if seg1 & 1:
            a = nrt - 1
            pltpu.sync_copy(src_hbm.at[pl.ds(i0 + a, 1)], lsidx)
            pltpu.sync_copy(dst_hbm.at[pl.ds(i0 + a, 1)], ldidx)
            pltpu.async_copy(g_hbm.at[lsidx.at[0]], rows.at[0], sem0)
            pltpu.make_async_copy(g_hbm.at[pl.ds(0, bch)],
                                  rows.at[0], sem0).wait()
            pltpu.sync_copy(rows.at[0], acc.at[ldidx.at[0]], add=True)

        if nlft:
            @pl.when(wid < nlft)
            def _():
                j0 = nw * nrt + wid
                pltpu.sync_copy(src_hbm.at[pl.ds(j0, 1)], lsidx)
                pltpu.sync_copy(dst_hbm.at[pl.ds(j0, 1)], ldidx)
                pltpu.async_copy(g_hbm.at[lsidx.at[0]], rows.at[0], sem0)
                pltpu.make_async_copy(g_hbm.at[pl.ds(0, bch)],
                                      rows.at[0], sem0).wait()
                pltpu.sync_copy(rows.at[0], acc.at[ldidx.at[0]], add=True)

        plsc.subcore_barrier()
        pltpu.sync_copy(acc.at[pl.ds(r0, rpt)],
                        out_hbm.at[cid, pl.ds(r0, rpt)])

    return k(g, src2, dst2, zslice)


def kernel(x, edge_index, W1, b1, W2, b2):
    ei = edge_index.astype(jnp.int32)
    src = ei[0]
    dst = ei[1]
    e = src.shape[0]
    src2 = src.reshape(e // 128, 128)
    dst2 = dst.reshape(e // 128, 128)

    degp = _sc_deg(dst)
    g1, dinv = _tc_mm_scale(degp, x, W1)

    z1 = jnp.zeros((N_NODES // NS, g1.shape[1]), g1.dtype)
    s1 = _sc_edge(g1, src2, dst2, z1)
    g2 = _tc_mid(s1, dinv, b1.reshape(1, -1), W2)

    z2 = jnp.zeros((N_NODES // NS, g2.shape[1]), g2.dtype)
    s2 = _sc_edge(g2, src2, dst2, z2)
    return _tc_final(s2, dinv, b2.reshape(1, -1))
